# Initial kernel scaffold; baseline (speedup 1.0000x reference)
import jax
import jax.numpy as jnp
from jax.experimental import pallas as pl


def kernel(points, faces, vert_tri_indices, vert_tri_weights):
    raise NotImplementedError("write your pallas kernel here")


# speedup vs baseline: 13.8109x; 13.8109x over previous
import dataclasses

import jax
import jax.numpy as jnp
from jax import lax
from jax.experimental import pallas as pl
from jax.experimental.pallas import tpu as pltpu
from jax.experimental.pallas import tpu_sc as plsc

_L = 16
_NT = 32

_F_CH = 896
_F_CHUNKS = 7
_F_IDX_ROWS = _F_CH // 128
_F_PER_TILE = _F_CH * _F_CHUNKS
_F_PAD = _F_PER_TILE * _NT

_V_CH = 448
_V_CHUNKS = 7
_V_IDX_ROWS = _V_CH * 8 // 128
_V_PER_TILE = _V_CH * _V_CHUNKS
_N_PAD = _V_PER_TILE * _NT


def _c16(v):
    return jnp.full((_L,), v, jnp.int32)


def _rsqrt(s):
    i = plsc.bitcast(s, jnp.int32)
    y = plsc.bitcast(jnp.int32(0x5F3759DF) - (i >> 1), jnp.float32)
    xh = s * 0.5
    for _ in range(3):
        y = y * (1.5 - xh * y * y)
    return y


def _wid():
    return lax.axis_index("s") * 2 + lax.axis_index("c")


def _face_body(p_hbm, f0, f1, f2, fn_hbm, idx0, idx1, idx2, r0, r1, r2, outb):
    wid = _wid()
    iota = lax.iota(jnp.int32, _L)

    @pl.loop(0, _F_CHUNKS)
    def _chunk(ch):
        cidx = wid * _F_CHUNKS + ch
        base = cidx * _F_CH
        pltpu.sync_copy(f0.at[pl.ds(base, _F_CH)], idx0)
        pltpu.sync_copy(f1.at[pl.ds(base, _F_CH)], idx1)
        pltpu.sync_copy(f2.at[pl.ds(base, _F_CH)], idx2)
        for k in range(_F_IDX_ROWS):
            sl = pl.ds(k * 128, 128)
            pltpu.sync_copy(p_hbm.at[idx0.at[sl]], r0.at[sl])
            pltpu.sync_copy(p_hbm.at[idx1.at[sl]], r1.at[sl])
            pltpu.sync_copy(p_hbm.at[idx2.at[sl]], r2.at[sl])

        @pl.loop(0, _F_CH // _L)
        def _grp(g):
            riv = g * _L + iota
            a0 = [plsc.load_gather(r0, [riv, _c16(j)]) for j in range(12)]
            a1 = [plsc.load_gather(r1, [riv, _c16(j)]) for j in range(12)]
            a2 = [plsc.load_gather(r2, [riv, _c16(j)]) for j in range(12)]
            u = [a1[j] - a0[j] for j in range(12)]
            v = [a2[j] - a0[j] for j in range(12)]
            for b in range(4):
                X, Y, Z = 3 * b, 3 * b + 1, 3 * b + 2
                nx = u[Y] * v[Z] - u[Z] * v[Y]
                ny = u[Z] * v[X] - u[X] * v[Z]
                nz = u[X] * v[Y] - u[Y] * v[X]
                s = jnp.maximum(nx * nx + ny * ny + nz * nz, 1e-24)
                r = _rsqrt(s)
                plsc.store_scatter(outb, [riv, _c16(X)], nx * r)
                plsc.store_scatter(outb, [riv, _c16(Y)], ny * r)
                plsc.store_scatter(outb, [riv, _c16(Z)], nz * r)

        pltpu.sync_copy(outb, fn_hbm.at[pl.ds(base, _F_CH)])


def _vert_body(fn_hbm, vt, w_hbm, vn_hbm, idxb, wb, rows, outb):
    wid = _wid()
    iota = lax.iota(jnp.int32, _L)

    @pl.loop(0, _V_CHUNKS)
    def _chunk(ch):
        cidx = wid * _V_CHUNKS + ch
        pltpu.sync_copy(vt.at[pl.ds(cidx * _V_CH * 8, _V_CH * 8)], idxb)
        pltpu.sync_copy(w_hbm.at[pl.ds(cidx * _V_CH * 8, _V_CH * 8)], wb)
        for k in range(_V_IDX_ROWS):
            sl = pl.ds(k * 128, 128)
            pltpu.sync_copy(fn_hbm.at[idxb.at[sl]], rows.at[sl])

        @pl.loop(0, _V_CH // _L)
        def _grp(g):
            riv = g * _L + iota
            rb = riv * 8
            rix = [rb + c for c in range(8)]
            ws = [plsc.load_gather(wb, [rix[c]]) for c in range(8)]
            for b in range(4):
                comp = []
                for k in range(3):
                    cj = _c16(3 * b + k)
                    t = ws[0] * plsc.load_gather(rows, [rix[0], cj])
                    for c in range(1, 8):
                        t = t + ws[c] * plsc.load_gather(rows, [rix[c], cj])
                    comp.append(t)
                s = jnp.maximum(
                    comp[0] * comp[0] + comp[1] * comp[1] + comp[2] * comp[2],
                    1e-24,
                )
                r = _rsqrt(s)
                for k in range(3):
                    plsc.store_scatter(outb, [riv, _c16(3 * b + k)], comp[k] * r)

        pltpu.sync_copy(outb, vn_hbm.at[pl.ds(cidx * _V_CH, _V_CH)])


def _center_body(x_ref, o_ref, n):
    x = x_ref[...]
    s = jnp.sum(x, axis=0, keepdims=True)
    m = s[:, 0:16]
    for k in range(1, 8):
        m = m + s[:, 16 * k:16 * (k + 1)]
    m = m * (1.0 / n)
    o_ref[...] = x - jnp.concatenate([m] * 8, axis=1)


def kernel(points, faces, vert_tri_indices, vert_tri_weights):
    bs, n, _ = points.shape
    f = faces.shape[0]
    dt = points.dtype

    p_rows = jnp.transpose(points, (1, 0, 2)).reshape(n, bs * 3)
    p_rows = jnp.pad(p_rows, ((0, 0), (0, 16 - bs * 3)))
    f1d = [jnp.pad(faces[:, k], (0, _F_PAD - f)) for k in range(3)]
    vt1d = jnp.pad(vert_tri_indices.reshape(-1), (0, (_N_PAD - n) * 8))
    w_flat = jnp.pad(vert_tri_weights.reshape(-1), (0, (_N_PAD - n) * 8))

    mesh = plsc.VectorSubcoreMesh(core_axis_name="c", subcore_axis_name="s")
    f32 = jnp.float32
    i32 = jnp.int32
    cp = pltpu.CompilerParams()
    fields = pltpu.CompilerParams.__dataclass_fields__
    if "needs_layout_passes" in fields:
        cp = dataclasses.replace(cp, needs_layout_passes=False)
    if "use_tc_tiling_on_sc" in fields:
        cp = dataclasses.replace(cp, use_tc_tiling_on_sc=False)

    fn_rows = pl.kernel(
        _face_body,
        out_type=jax.ShapeDtypeStruct((_F_PAD, 16), f32),
        mesh=mesh,
        scratch_types=[
            pltpu.VMEM((_F_CH,), i32),
            pltpu.VMEM((_F_CH,), i32),
            pltpu.VMEM((_F_CH,), i32),
            pltpu.VMEM((_F_CH, 16), f32),
            pltpu.VMEM((_F_CH, 16), f32),
            pltpu.VMEM((_F_CH, 16), f32),
            pltpu.VMEM((_F_CH, 16), f32),
        ],
        compiler_params=cp,
    )(p_rows, f1d[0], f1d[1], f1d[2])

    vn_rows = pl.kernel(
        _vert_body,
        out_type=jax.ShapeDtypeStruct((_N_PAD, 16), f32),
        mesh=mesh,
        scratch_types=[
            pltpu.VMEM((_V_CH * 8,), i32),
            pltpu.VMEM((_V_CH * 8,), f32),
            pltpu.VMEM((_V_CH * 8, 16), f32),
            pltpu.VMEM((_V_CH, 16), f32),
        ],
        compiler_params=cp,
    )(fn_rows, vt1d, w_flat)

    pts2d = pl.pallas_call(
        lambda x_ref, o_ref: _center_body(x_ref, o_ref, n),
        out_shape=jax.ShapeDtypeStruct((n // 8, 128), f32),
    )(p_rows.reshape(n // 8, 128))

    pts = pts2d.reshape(n, 16)[:, : bs * 3].reshape(n, bs, 3).transpose(1, 0, 2)
    vn = vn_rows[:n, : bs * 3].reshape(n, bs, 3).transpose(1, 0, 2)
    rgb = jnp.ones((bs, n, 3), dt)
    return pts, vn, rgb


# speedup vs baseline: 18.3205x; 1.3265x over previous
import dataclasses

import jax
import jax.numpy as jnp
from jax import lax
from jax.experimental import pallas as pl
from jax.experimental.pallas import tpu as pltpu
from jax.experimental.pallas import tpu_sc as plsc

_L = 16
_NT = 32

_F_CH = 896
_F_CHUNKS = 7
_F_IDX_ROWS = _F_CH // 128
_F_PER_TILE = _F_CH * _F_CHUNKS
_F_PAD = _F_PER_TILE * _NT

_V_CH = 448
_V_CHUNKS = 7
_V_IDX_ROWS = _V_CH * 8 // 128
_V_PER_TILE = _V_CH * _V_CHUNKS
_N_PAD = _V_PER_TILE * _NT


def _c16(v):
    return jnp.full((_L,), v, jnp.int32)


def _rsqrt(s):
    i = plsc.bitcast(s, jnp.int32)
    y = plsc.bitcast(jnp.int32(0x5F3759DF) - (i >> 1), jnp.float32)
    xh = s * 0.5
    for _ in range(3):
        y = y * (1.5 - xh * y * y)
    return y


def _wid():
    return lax.axis_index("s") * 2 + lax.axis_index("c")


def _face_body(p_hbm, f0, f1, f2, fn_hbm, idx0, idx1, idx2, r0, r1, r2, outb,
               sem):
    wid = _wid()
    iota = lax.iota(jnp.int32, _L)

    @pl.loop(0, _F_CHUNKS)
    def _chunk(ch):
        cidx = wid * _F_CHUNKS + ch
        base = cidx * _F_CH
        cps = [
            pltpu.async_copy(f0.at[pl.ds(base, _F_CH)], idx0, sem),
            pltpu.async_copy(f1.at[pl.ds(base, _F_CH)], idx1, sem),
            pltpu.async_copy(f2.at[pl.ds(base, _F_CH)], idx2, sem),
        ]
        for c in cps:
            c.wait()
        cps = []
        for k in range(_F_IDX_ROWS):
            sl = pl.ds(k * 128, 128)
            cps.append(pltpu.async_copy(p_hbm.at[idx0.at[sl]], r0.at[sl], sem))
            cps.append(pltpu.async_copy(p_hbm.at[idx1.at[sl]], r1.at[sl], sem))
            cps.append(pltpu.async_copy(p_hbm.at[idx2.at[sl]], r2.at[sl], sem))
        for c in cps:
            c.wait()

        @pl.loop(0, _F_CH // _L)
        def _grp(g):
            riv = g * _L + iota
            a0 = [plsc.load_gather(r0, [riv, _c16(j)]) for j in range(12)]
            a1 = [plsc.load_gather(r1, [riv, _c16(j)]) for j in range(12)]
            a2 = [plsc.load_gather(r2, [riv, _c16(j)]) for j in range(12)]
            u = [a1[j] - a0[j] for j in range(12)]
            v = [a2[j] - a0[j] for j in range(12)]
            for b in range(4):
                X, Y, Z = 3 * b, 3 * b + 1, 3 * b + 2
                nx = u[Y] * v[Z] - u[Z] * v[Y]
                ny = u[Z] * v[X] - u[X] * v[Z]
                nz = u[X] * v[Y] - u[Y] * v[X]
                s = jnp.maximum(nx * nx + ny * ny + nz * nz, 1e-24)
                r = _rsqrt(s)
                plsc.store_scatter(outb, [riv, _c16(X)], nx * r)
                plsc.store_scatter(outb, [riv, _c16(Y)], ny * r)
                plsc.store_scatter(outb, [riv, _c16(Z)], nz * r)

        pltpu.sync_copy(outb, fn_hbm.at[pl.ds(base, _F_CH)])


def _vert_body(fn_hbm, vt, w_hbm, vn_hbm, idxb, wb, rows, outb, sem):
    wid = _wid()
    iota = lax.iota(jnp.int32, _L)

    @pl.loop(0, _V_CHUNKS)
    def _chunk(ch):
        cidx = wid * _V_CHUNKS + ch
        cps = [
            pltpu.async_copy(
                vt.at[pl.ds(cidx * _V_CH * 8, _V_CH * 8)], idxb, sem
            ),
            pltpu.async_copy(
                w_hbm.at[pl.ds(cidx * _V_CH * 8, _V_CH * 8)], wb, sem
            ),
        ]
        for c in cps:
            c.wait()
        cps = []
        for k in range(_V_IDX_ROWS):
            sl = pl.ds(k * 128, 128)
            cps.append(pltpu.async_copy(fn_hbm.at[idxb.at[sl]], rows.at[sl], sem))
        for c in cps:
            c.wait()

        @pl.loop(0, _V_CH // _L)
        def _grp(g):
            riv = g * _L + iota
            rb = riv * 8
            rix = [rb + c for c in range(8)]
            ws = [plsc.load_gather(wb, [rix[c]]) for c in range(8)]
            for b in range(4):
                comp = []
                for k in range(3):
                    cj = _c16(3 * b + k)
                    t = ws[0] * plsc.load_gather(rows, [rix[0], cj])
                    for c in range(1, 8):
                        t = t + ws[c] * plsc.load_gather(rows, [rix[c], cj])
                    comp.append(t)
                s = jnp.maximum(
                    comp[0] * comp[0] + comp[1] * comp[1] + comp[2] * comp[2],
                    1e-24,
                )
                r = _rsqrt(s)
                for k in range(3):
                    plsc.store_scatter(outb, [riv, _c16(3 * b + k)], comp[k] * r)

        pltpu.sync_copy(outb, vn_hbm.at[pl.ds(cidx * _V_CH, _V_CH)])


def _center_body(x_ref, o_ref, n):
    x = x_ref[...]
    s = jnp.sum(x, axis=0, keepdims=True)
    m = s[:, 0:16]
    for k in range(1, 8):
        m = m + s[:, 16 * k:16 * (k + 1)]
    m = m * (1.0 / n)
    o_ref[...] = x - jnp.concatenate([m] * 8, axis=1)


def kernel(points, faces, vert_tri_indices, vert_tri_weights):
    bs, n, _ = points.shape
    f = faces.shape[0]
    dt = points.dtype

    p_rows = jnp.transpose(points, (1, 0, 2)).reshape(n, bs * 3)
    p_rows = jnp.pad(p_rows, ((0, 0), (0, 16 - bs * 3)))
    f1d = [jnp.pad(faces[:, k], (0, _F_PAD - f)) for k in range(3)]
    vt1d = jnp.pad(vert_tri_indices.reshape(-1), (0, (_N_PAD - n) * 8))
    w_flat = jnp.pad(vert_tri_weights.reshape(-1), (0, (_N_PAD - n) * 8))

    mesh = plsc.VectorSubcoreMesh(core_axis_name="c", subcore_axis_name="s")
    f32 = jnp.float32
    i32 = jnp.int32
    cp = pltpu.CompilerParams()
    fields = pltpu.CompilerParams.__dataclass_fields__
    if "needs_layout_passes" in fields:
        cp = dataclasses.replace(cp, needs_layout_passes=False)
    if "use_tc_tiling_on_sc" in fields:
        cp = dataclasses.replace(cp, use_tc_tiling_on_sc=False)

    fn_rows = pl.kernel(
        _face_body,
        out_type=jax.ShapeDtypeStruct((_F_PAD, 16), f32),
        mesh=mesh,
        scratch_types=[
            pltpu.VMEM((_F_CH,), i32),
            pltpu.VMEM((_F_CH,), i32),
            pltpu.VMEM((_F_CH,), i32),
            pltpu.VMEM((_F_CH, 16), f32),
            pltpu.VMEM((_F_CH, 16), f32),
            pltpu.VMEM((_F_CH, 16), f32),
            pltpu.VMEM((_F_CH, 16), f32),
            pltpu.SemaphoreType.DMA,
        ],
        compiler_params=cp,
    )(p_rows, f1d[0], f1d[1], f1d[2])

    vn_rows = pl.kernel(
        _vert_body,
        out_type=jax.ShapeDtypeStruct((_N_PAD, 16), f32),
        mesh=mesh,
        scratch_types=[
            pltpu.VMEM((_V_CH * 8,), i32),
            pltpu.VMEM((_V_CH * 8,), f32),
            pltpu.VMEM((_V_CH * 8, 16), f32),
            pltpu.VMEM((_V_CH, 16), f32),
            pltpu.SemaphoreType.DMA,
        ],
        compiler_params=cp,
    )(fn_rows, vt1d, w_flat)

    pts2d = pl.pallas_call(
        lambda x_ref, o_ref: _center_body(x_ref, o_ref, n),
        out_shape=jax.ShapeDtypeStruct((n // 8, 128), f32),
    )(p_rows.reshape(n // 8, 128))

    pts = pts2d.reshape(n, 16)[:, : bs * 3].reshape(n, bs, 3).transpose(1, 0, 2)
    vn = vn_rows[:n, : bs * 3].reshape(n, bs, 3).transpose(1, 0, 2)
    rgb = jnp.ones((bs, n, 3), dt)
    return pts, vn, rgb


# speedup vs baseline: 18.3208x; 1.0000x over previous
import dataclasses

import jax
import jax.numpy as jnp
from jax import lax
from jax.experimental import pallas as pl
from jax.experimental.pallas import tpu as pltpu
from jax.experimental.pallas import tpu_sc as plsc

_L = 16
_NT = 32

_F_CH = 896
_F_CHUNKS = 7
_F_IDX_ROWS = _F_CH // 128
_F_PER_TILE = _F_CH * _F_CHUNKS
_F_PAD = _F_PER_TILE * _NT

_V_CH = 448
_V_CHUNKS = 7
_V_IDX_ROWS = _V_CH * 8 // 128
_V_PER_TILE = _V_CH * _V_CHUNKS
_N_PAD = _V_PER_TILE * _NT


def _c16(v):
    return jnp.full((_L,), v, jnp.int32)


def _rsqrt(s):
    i = plsc.bitcast(s, jnp.int32)
    y = plsc.bitcast(jnp.int32(0x5F3759DF) - (i >> 1), jnp.float32)
    xh = s * 0.5
    for _ in range(3):
        y = y * (1.5 - xh * y * y)
    return y


def _wid():
    return lax.axis_index("s") * 2 + lax.axis_index("c")


def _face_body(p_hbm, f0, f1, f2, fn_hbm, idx0, idx1, idx2, r0, r1, r2, outb,
               sem):
    wid = _wid()
    iota = lax.iota(jnp.int32, _L)

    @pl.loop(0, _F_CHUNKS)
    def _chunk(ch):
        cidx = wid * _F_CHUNKS + ch
        base = cidx * _F_CH
        cps = [
            pltpu.async_copy(f0.at[pl.ds(base, _F_CH)], idx0, sem),
            pltpu.async_copy(f1.at[pl.ds(base, _F_CH)], idx1, sem),
            pltpu.async_copy(f2.at[pl.ds(base, _F_CH)], idx2, sem),
        ]
        for c in cps:
            c.wait()
        cps = [
            pltpu.async_copy(p_hbm.at[idx0], r0, sem),
            pltpu.async_copy(p_hbm.at[idx1], r1, sem),
            pltpu.async_copy(p_hbm.at[idx2], r2, sem),
        ]
        for c in cps:
            c.wait()

        @pl.loop(0, _F_CH // _L)
        def _grp(g):
            riv = g * _L + iota
            a0 = [plsc.load_gather(r0, [riv, _c16(j)]) for j in range(12)]
            a1 = [plsc.load_gather(r1, [riv, _c16(j)]) for j in range(12)]
            a2 = [plsc.load_gather(r2, [riv, _c16(j)]) for j in range(12)]
            u = [a1[j] - a0[j] for j in range(12)]
            v = [a2[j] - a0[j] for j in range(12)]
            for b in range(4):
                X, Y, Z = 3 * b, 3 * b + 1, 3 * b + 2
                nx = u[Y] * v[Z] - u[Z] * v[Y]
                ny = u[Z] * v[X] - u[X] * v[Z]
                nz = u[X] * v[Y] - u[Y] * v[X]
                s = jnp.maximum(nx * nx + ny * ny + nz * nz, 1e-24)
                r = _rsqrt(s)
                plsc.store_scatter(outb, [riv, _c16(X)], nx * r)
                plsc.store_scatter(outb, [riv, _c16(Y)], ny * r)
                plsc.store_scatter(outb, [riv, _c16(Z)], nz * r)

        pltpu.sync_copy(outb, fn_hbm.at[pl.ds(base, _F_CH)])


def _vert_body(fn_hbm, vt, w_hbm, vn_hbm, idxb, wb, rows, outb, sem):
    wid = _wid()
    iota = lax.iota(jnp.int32, _L)

    @pl.loop(0, _V_CHUNKS)
    def _chunk(ch):
        cidx = wid * _V_CHUNKS + ch
        cps = [
            pltpu.async_copy(
                vt.at[pl.ds(cidx * _V_CH * 8, _V_CH * 8)], idxb, sem
            ),
            pltpu.async_copy(
                w_hbm.at[pl.ds(cidx * _V_CH * 8, _V_CH * 8)], wb, sem
            ),
        ]
        for c in cps:
            c.wait()
        pltpu.async_copy(fn_hbm.at[idxb], rows, sem).wait()

        @pl.loop(0, _V_CH // _L)
        def _grp(g):
            riv = g * _L + iota
            rb = riv * 8
            rix = [rb + c for c in range(8)]
            ws = [plsc.load_gather(wb, [rix[c]]) for c in range(8)]
            for b in range(4):
                comp = []
                for k in range(3):
                    cj = _c16(3 * b + k)
                    t = ws[0] * plsc.load_gather(rows, [rix[0], cj])
                    for c in range(1, 8):
                        t = t + ws[c] * plsc.load_gather(rows, [rix[c], cj])
                    comp.append(t)
                s = jnp.maximum(
                    comp[0] * comp[0] + comp[1] * comp[1] + comp[2] * comp[2],
                    1e-24,
                )
                r = _rsqrt(s)
                for k in range(3):
                    plsc.store_scatter(outb, [riv, _c16(3 * b + k)], comp[k] * r)

        pltpu.sync_copy(outb, vn_hbm.at[pl.ds(cidx * _V_CH, _V_CH)])


def _center_body(x_ref, o_ref, n):
    x = x_ref[...]
    s = jnp.sum(x, axis=0, keepdims=True)
    m = s[:, 0:16]
    for k in range(1, 8):
        m = m + s[:, 16 * k:16 * (k + 1)]
    m = m * (1.0 / n)
    o_ref[...] = x - jnp.concatenate([m] * 8, axis=1)


def kernel(points, faces, vert_tri_indices, vert_tri_weights):
    bs, n, _ = points.shape
    f = faces.shape[0]
    dt = points.dtype

    p_rows = jnp.transpose(points, (1, 0, 2)).reshape(n, bs * 3)
    p_rows = jnp.pad(p_rows, ((0, 0), (0, 16 - bs * 3)))
    f1d = [jnp.pad(faces[:, k], (0, _F_PAD - f)) for k in range(3)]
    vt1d = jnp.pad(vert_tri_indices.reshape(-1), (0, (_N_PAD - n) * 8))
    w_flat = jnp.pad(vert_tri_weights.reshape(-1), (0, (_N_PAD - n) * 8))

    mesh = plsc.VectorSubcoreMesh(core_axis_name="c", subcore_axis_name="s")
    f32 = jnp.float32
    i32 = jnp.int32
    cp = pltpu.CompilerParams()
    fields = pltpu.CompilerParams.__dataclass_fields__
    if "needs_layout_passes" in fields:
        cp = dataclasses.replace(cp, needs_layout_passes=False)
    if "use_tc_tiling_on_sc" in fields:
        cp = dataclasses.replace(cp, use_tc_tiling_on_sc=False)

    fn_rows = pl.kernel(
        _face_body,
        out_type=jax.ShapeDtypeStruct((_F_PAD, 16), f32),
        mesh=mesh,
        scratch_types=[
            pltpu.VMEM((_F_CH,), i32),
            pltpu.VMEM((_F_CH,), i32),
            pltpu.VMEM((_F_CH,), i32),
            pltpu.VMEM((_F_CH, 16), f32),
            pltpu.VMEM((_F_CH, 16), f32),
            pltpu.VMEM((_F_CH, 16), f32),
            pltpu.VMEM((_F_CH, 16), f32),
            pltpu.SemaphoreType.DMA,
        ],
        compiler_params=cp,
    )(p_rows, f1d[0], f1d[1], f1d[2])

    vn_rows = pl.kernel(
        _vert_body,
        out_type=jax.ShapeDtypeStruct((_N_PAD, 16), f32),
        mesh=mesh,
        scratch_types=[
            pltpu.VMEM((_V_CH * 8,), i32),
            pltpu.VMEM((_V_CH * 8,), f32),
            pltpu.VMEM((_V_CH * 8, 16), f32),
            pltpu.VMEM((_V_CH, 16), f32),
            pltpu.SemaphoreType.DMA,
        ],
        compiler_params=cp,
    )(fn_rows, vt1d, w_flat)

    pts2d = pl.pallas_call(
        lambda x_ref, o_ref: _center_body(x_ref, o_ref, n),
        out_shape=jax.ShapeDtypeStruct((n // 8, 128), f32),
    )(p_rows.reshape(n // 8, 128))

    pts = pts2d.reshape(n, 16)[:, : bs * 3].reshape(n, bs, 3).transpose(1, 0, 2)
    vn = vn_rows[:n, : bs * 3].reshape(n, bs, 3).transpose(1, 0, 2)
    rgb = jnp.ones((bs, n, 3), dt)
    return pts, vn, rgb
